# Initial kernel scaffold; baseline (speedup 1.0000x reference)
import jax
import jax.numpy as jnp
from jax.experimental import pallas as pl


def kernel(x, t_range, ew1, eb1, ew2, eb2, ew3, eb3, ew4, eb4, pw, dw1, db1, dw2, db2, dw3, db3, dw4, db4):
    raise NotImplementedError("write your pallas kernel here")


# speedup vs baseline: 1.8805x; 1.8805x over previous
import jax
import jax.numpy as jnp
from jax.experimental import pallas as pl
from jax.experimental.pallas import tpu as pltpu

LANES = 128
GROUP = 32
NPACK = LANES // GROUP


def _make_body(tb, G, offs):
    (o_ew1, o_eb1, o_ew2, o_eb2, o_ew3, o_eb3, o_ew4, o_eb4,
     o_pw, o_dw1, o_db1, o_dw2, o_db2, o_dw3, o_db3, o_dw4, o_db4) = offs
    P = LANES
    f32 = jnp.float32

    def body(x0_ref, tp_ref, slab_ref, out_ref):
        def W(o):
            return slab_ref[o:o + P, :]

        def Bv(o):
            return slab_ref[o:o + 1, :]

        h = x0_ref[...]
        h = jnp.maximum(jnp.dot(h, W(o_ew1), preferred_element_type=f32) + Bv(o_eb1), 0.0)
        h = jnp.maximum(jnp.dot(h, W(o_ew2), preferred_element_type=f32) + Bv(o_eb2), 0.0)
        h = jnp.maximum(jnp.dot(h, W(o_ew3), preferred_element_type=f32) + Bv(o_eb3), 0.0)
        z0 = jnp.tanh(jnp.dot(h, W(o_ew4), preferred_element_type=f32) + Bv(o_eb4))

        poly = jnp.dot(tp_ref[...], W(o_pw), preferred_element_type=f32)

        z = (z0[:, None, :] + poly[None, :, :]).reshape(tb * G, P)

        h = jnp.maximum(jnp.dot(z, W(o_dw1), preferred_element_type=f32) + Bv(o_db1), 0.0)
        h = jnp.maximum(jnp.dot(h, W(o_dw2), preferred_element_type=f32) + Bv(o_db2), 0.0)
        h = jnp.maximum(jnp.dot(h, W(o_dw3), preferred_element_type=f32) + Bv(o_db3), 0.0)
        y = jnp.tanh(jnp.dot(h, W(o_dw4), preferred_element_type=f32) + Bv(o_db4))

        yr = y.reshape(tb, G, P)
        out_ref[...] = jnp.concatenate(
            [yr[:, g, :16] for g in range(G)], axis=-1).astype(out_ref.dtype)

    return body


def _pad(w, rows, cols):
    return jnp.zeros((rows, cols), jnp.float32).at[:w.shape[0], :w.shape[1]].set(
        w.astype(jnp.float32))


def _block_diag4(w):
    wp = _pad(w, GROUP, GROUP)
    z = jnp.zeros((GROUP, GROUP), jnp.float32)
    rows = []
    for j in range(NPACK):
        rows.append(jnp.concatenate(
            [wp if i == j else z for i in range(NPACK)], axis=1))
    return jnp.concatenate(rows, axis=0)


def _lane_rep4(wp):
    return jnp.concatenate([wp] * NPACK, axis=1)


def _pack_slab(blocks):
    out, offsets, off = [], [], 0
    for b in blocks:
        nrows = b.shape[0]
        out.append(b)
        offsets.append(off)
        off += nrows
    return jnp.concatenate(out, axis=0), tuple(offsets)


def kernel(x, t_range, ew1, eb1, ew2, eb2, ew3, eb3, ew4, eb4, pw,
           dw1, db1, dw2, db2, dw3, db3, dw4, db4):
    B, T, Q = x.shape
    P = LANES
    G = T // NPACK
    degree = pw.shape[0]
    L = pw.shape[1]

    tb = 256 if B >= 512 else max(8, min(B, 128))
    n_tiles = -(-B // tb)
    Bp = n_tiles * tb

    x2d = x.reshape(B, T * Q).astype(jnp.float32)
    if T * Q != P:
        x2d = _pad(x2d, B, P)[:, :P] if T * Q < P else x2d[:, :P]
    if Bp != B:
        x2d = jnp.zeros((Bp, P), jnp.float32).at[:B, :].set(x2d)

    t_col = t_range.astype(jnp.float32).reshape(T, 1)
    powers = jnp.concatenate([t_col ** i for i in range(1, degree + 1)], axis=1)
    tp = jnp.zeros((T, GROUP), jnp.float32).at[:, :degree].set(powers)
    tp = tp.reshape(G, NPACK * GROUP)

    dw4p = jnp.zeros((P, P), jnp.float32)
    dw4c = dw4.astype(jnp.float32)
    db4p = jnp.zeros((8, P), jnp.float32)
    for j in range(NPACK):
        dw4p = dw4p.at[j * GROUP:j * GROUP + dw4.shape[0], j * Q:(j + 1) * Q].set(dw4c)
        db4p = db4p.at[:1, j * Q:(j + 1) * Q].set(db4.astype(jnp.float32))

    blocks = [
        _pad(ew1, P, P), _pad(eb1, 8, P),
        _pad(ew2, P, P), _pad(eb2, 8, P),
        _pad(ew3, P, P), _pad(eb3, 8, P),
        _lane_rep4(_pad(ew4, P, GROUP)), _lane_rep4(_pad(eb4, 8, GROUP)),
        _block_diag4(pw),
        _block_diag4(dw1), _lane_rep4(_pad(db1, 8, GROUP)),
        _block_diag4(dw2), _lane_rep4(_pad(db2, 8, GROUP)),
        _block_diag4(dw3), _lane_rep4(_pad(db3, 8, GROUP)),
        dw4p, db4p,
    ]
    slab, offs = _pack_slab(blocks)
    slab_rows = slab.shape[0]

    body = _make_body(tb, G, offs)

    out = pl.pallas_call(
        body,
        out_shape=jax.ShapeDtypeStruct((Bp, P), jnp.float32),
        grid=(n_tiles,),
        in_specs=[
            pl.BlockSpec((tb, P), lambda b: (b, 0)),
            pl.BlockSpec((G, P), lambda b: (0, 0)),
            pl.BlockSpec((slab_rows, P), lambda b: (0, 0)),
        ],
        out_specs=pl.BlockSpec((tb, P), lambda b: (b, 0)),
        compiler_params=pltpu.CompilerParams(dimension_semantics=("parallel",)),
    )(x2d, tp, slab)

    return out[:B].reshape(B, T, Q)
